# baseline (speedup 1.0000x reference)
import numpy as np
import jax
import jax.numpy as jnp
from jax.experimental import pallas as pl

_ANCHOR_W = (116.0, 156.0, 373.0)
_ANCHOR_H = (90.0, 198.0, 326.0)
_NG = 64
_NCELL = _NG * _NG
_ATTRS_IN = 87
_ATTRS_OUT = 86
_STRIDE = 512.0 / _NG


def _decode_body(x_ref, o_ref):
    a = pl.program_id(1)
    t = x_ref[0]

    s = jax.nn.sigmoid(t)

    gx = jax.lax.broadcasted_iota(jnp.int32, (_NG, _NG), 1).astype(jnp.float32)
    gy = jax.lax.broadcasted_iota(jnp.int32, (_NG, _NG), 0).astype(jnp.float32)

    aw = jnp.where(a == 0, _ANCHOR_W[0], jnp.where(a == 1, _ANCHOR_W[1], _ANCHOR_W[2]))
    ah = jnp.where(a == 0, _ANCHOR_H[0], jnp.where(a == 1, _ANCHOR_H[1], _ANCHOR_H[2]))

    px = (s[0] + gx) * _STRIDE
    py = (s[1] + gy) * _STRIDE
    pw = jnp.exp(t[2]) * aw
    plh = jnp.exp(t[3]) * ah
    theta = jnp.arctan2(jnp.tanh(t[4]), jnp.tanh(t[5])) * (90.0 / np.pi)
    zero = jnp.zeros((_NG, _NG), jnp.float32)

    five = jnp.stack([px, py, pw, plh, theta, zero, zero, zero], axis=0)

    rA = jax.lax.broadcasted_iota(jnp.int32, (8, _ATTRS_OUT), 0)
    cA = jax.lax.broadcasted_iota(jnp.int32, (8, _ATTRS_OUT), 1)
    selA = jnp.where((cA < 5) & (rA == cA), 1.0, 0.0).astype(jnp.bfloat16)
    rB = jax.lax.broadcasted_iota(jnp.int32, (_ATTRS_IN, _ATTRS_OUT), 0)
    cB = jax.lax.broadcasted_iota(jnp.int32, (_ATTRS_IN, _ATTRS_OUT), 1)
    selB = jnp.where((cB >= 5) & (rB == cB + 1), 1.0, 0.0).astype(jnp.bfloat16)

    dims = (((0,), (0,)), ((), ()))
    hi = five.astype(jnp.bfloat16)
    lo = (five - hi.astype(jnp.float32)).astype(jnp.bfloat16)
    out = (
        jax.lax.dot_general(s.astype(jnp.bfloat16), selB, dims,
                            preferred_element_type=jnp.float32)
        + jax.lax.dot_general(hi, selA, dims,
                              preferred_element_type=jnp.float32)
        + jax.lax.dot_general(lo, selA, dims,
                              preferred_element_type=jnp.float32)
    )
    o_ref[0] = out.reshape(_NCELL, _ATTRS_OUT)


def kernel(x):
    nB = x.shape[0]
    out_shape = jax.ShapeDtypeStruct((nB, 3 * _NCELL, _ATTRS_OUT), jnp.float32)
    return pl.pallas_call(
        _decode_body,
        grid=(nB, 3),
        in_specs=[
            pl.BlockSpec((1, _ATTRS_IN, _NG, _NG), lambda b, a: (b, a, 0, 0)),
        ],
        out_specs=pl.BlockSpec((1, _NCELL, _ATTRS_OUT), lambda b, a: (b, a, 0)),
        out_shape=out_shape,
    )(x)


# speedup vs baseline: 1.0161x; 1.0161x over previous
import numpy as np
import jax
import jax.numpy as jnp
from jax.experimental import pallas as pl

_ANCHOR_W = (116.0, 156.0, 373.0)
_ANCHOR_H = (90.0, 198.0, 326.0)
_NG = 64
_NCELL = _NG * _NG
_ATTRS_IN = 87
_ATTRS_OUT = 86
_STRIDE = 512.0 / _NG


def _decode_body(x_ref, o_ref):
    a = pl.program_id(1)
    t = x_ref[0]

    s = jax.nn.sigmoid(t)

    gx = jax.lax.broadcasted_iota(jnp.int32, (_NG, _NG), 1).astype(jnp.float32)
    gy = jax.lax.broadcasted_iota(jnp.int32, (_NG, _NG), 0).astype(jnp.float32)

    aw = jnp.where(a == 0, _ANCHOR_W[0], jnp.where(a == 1, _ANCHOR_W[1], _ANCHOR_W[2]))
    ah = jnp.where(a == 0, _ANCHOR_H[0], jnp.where(a == 1, _ANCHOR_H[1], _ANCHOR_H[2]))

    px = (s[0] + gx) * _STRIDE
    py = (s[1] + gy) * _STRIDE
    pw = jnp.exp(t[2]) * aw
    plh = jnp.exp(t[3]) * ah
    theta = jnp.arctan2(jnp.tanh(t[4]), jnp.tanh(t[5])) * (90.0 / np.pi)
    zero = jnp.zeros((_NG, _NG), jnp.float32)

    five = jnp.stack([px, py, pw, plh, theta, zero, zero, zero], axis=0)

    rA = jax.lax.broadcasted_iota(jnp.int32, (8, _ATTRS_OUT), 0)
    cA = jax.lax.broadcasted_iota(jnp.int32, (8, _ATTRS_OUT), 1)
    selA = jnp.where((cA < 5) & (rA == cA), 1.0, 0.0).astype(jnp.bfloat16)
    rB = jax.lax.broadcasted_iota(jnp.int32, (_ATTRS_IN, _ATTRS_OUT), 0)
    cB = jax.lax.broadcasted_iota(jnp.int32, (_ATTRS_IN, _ATTRS_OUT), 1)
    selB = jnp.where((cB >= 5) & (rB == cB + 1), 1.0, 0.0).astype(jnp.bfloat16)

    dims = (((0,), (0,)), ((), ()))
    hi = five.astype(jnp.bfloat16)
    lo = (five - hi.astype(jnp.float32)).astype(jnp.bfloat16)
    out = (
        jax.lax.dot_general(s.astype(jnp.bfloat16), selB, dims,
                            preferred_element_type=jnp.float32)
        + jax.lax.dot_general(hi, selA, dims,
                              preferred_element_type=jnp.float32)
        + jax.lax.dot_general(lo, selA, dims,
                              preferred_element_type=jnp.float32)
    )
    o_ref[0, 0] = out.reshape(_NCELL, _ATTRS_OUT)


def kernel(x):
    nB = x.shape[0]
    out_shape = jax.ShapeDtypeStruct((nB, 3, _NCELL, _ATTRS_OUT), jnp.float32)
    out = pl.pallas_call(
        _decode_body,
        grid=(nB, 3),
        in_specs=[
            pl.BlockSpec((1, _ATTRS_IN, _NG, _NG), lambda b, a: (b, a, 0, 0)),
        ],
        out_specs=pl.BlockSpec((1, 1, _NCELL, _ATTRS_OUT), lambda b, a: (b, a, 0, 0)),
        out_shape=out_shape,
    )(x)
    return out.reshape(nB, 3 * _NCELL, _ATTRS_OUT)


# speedup vs baseline: 1.4328x; 1.4101x over previous
import numpy as np
import jax
import jax.numpy as jnp
from jax.experimental import pallas as pl

_ANCHOR_W = (116.0, 156.0, 373.0)
_ANCHOR_H = (90.0, 198.0, 326.0)
_NG = 64
_NCELL = _NG * _NG
_NR = 32
_NL = 128
_ATTRS_IN = 87
_ATTRS_OUT = 86
_STRIDE = 512.0 / _NG


def _decode_body(x_ref, o_ref):
    a = pl.program_id(1)
    t = x_ref[0]

    s = jax.nn.sigmoid(t)

    li = jax.lax.broadcasted_iota(jnp.int32, (_NR, _NL), 1)
    ri = jax.lax.broadcasted_iota(jnp.int32, (_NR, _NL), 0)
    gx = (li % _NG).astype(jnp.float32)
    gy = (ri * 2 + li // _NG).astype(jnp.float32)

    aw = jnp.where(a == 0, _ANCHOR_W[0], jnp.where(a == 1, _ANCHOR_W[1], _ANCHOR_W[2]))
    ah = jnp.where(a == 0, _ANCHOR_H[0], jnp.where(a == 1, _ANCHOR_H[1], _ANCHOR_H[2]))

    px = (s[0] + gx) * _STRIDE
    py = (s[1] + gy) * _STRIDE
    pw = jnp.exp(t[2]) * aw
    plh = jnp.exp(t[3]) * ah
    theta = jnp.arctan2(jnp.tanh(t[4]), jnp.tanh(t[5])) * (90.0 / np.pi)
    zero = jnp.zeros((_NR, _NL), jnp.float32)

    five = jnp.stack([px, py, pw, plh, theta, zero, zero, zero], axis=0)

    rA = jax.lax.broadcasted_iota(jnp.int32, (8, _ATTRS_OUT), 0)
    cA = jax.lax.broadcasted_iota(jnp.int32, (8, _ATTRS_OUT), 1)
    selA = jnp.where((cA < 5) & (rA == cA), 1.0, 0.0).astype(jnp.bfloat16)
    rB = jax.lax.broadcasted_iota(jnp.int32, (_ATTRS_IN, _ATTRS_OUT), 0)
    cB = jax.lax.broadcasted_iota(jnp.int32, (_ATTRS_IN, _ATTRS_OUT), 1)
    selB = jnp.where((cB >= 5) & (rB == cB + 1), 1.0, 0.0).astype(jnp.bfloat16)

    dims = (((0,), (0,)), ((), ()))
    hi = five.astype(jnp.bfloat16)
    lo = (five - hi.astype(jnp.float32)).astype(jnp.bfloat16)
    out = (
        jax.lax.dot_general(s.astype(jnp.bfloat16), selB, dims,
                            preferred_element_type=jnp.float32)
        + jax.lax.dot_general(hi, selA, dims,
                              preferred_element_type=jnp.float32)
        + jax.lax.dot_general(lo, selA, dims,
                              preferred_element_type=jnp.float32)
    )
    o_ref[0] = out.reshape(_NCELL, _ATTRS_OUT)


def kernel(x):
    nB = x.shape[0]
    xv = x.reshape(nB, 3 * _ATTRS_IN, _NR, _NL)
    out_shape = jax.ShapeDtypeStruct((nB, 3 * _NCELL, _ATTRS_OUT), jnp.float32)
    return pl.pallas_call(
        _decode_body,
        grid=(nB, 3),
        in_specs=[
            pl.BlockSpec((1, _ATTRS_IN, _NR, _NL), lambda b, a: (b, a, 0, 0)),
        ],
        out_specs=pl.BlockSpec((1, _NCELL, _ATTRS_OUT), lambda b, a: (b, a, 0)),
        out_shape=out_shape,
    )(xv)
